# baseline (speedup 1.0000x reference)
import functools

import jax
import jax.numpy as jnp
from jax import lax
from jax.experimental import pallas as pl
from jax.experimental.pallas import tpu as pltpu
from jax.experimental.pallas import tpu_sc as plsc

B = 16384
D = 64
NC = 2
NS = 16
NW = NC * NS
BPW = B // NW
CHUNK = 128
NCHUNK = BPW // CHUNK
GROUPS = BPW // 16


def _gmf_kernel(utab_hbm, itab_hbm, uid_hbm, iid_hbm, w_hbm, b_hbm, out_hbm,
                uidx, iidx, urows, vrows, wvec, bvec, outv, sem):
    wid = lax.axis_index("s") * NC + lax.axis_index("c")

    pltpu.sync_copy(uid_hbm.at[pl.ds(wid * NCHUNK, NCHUNK)], uidx)
    pltpu.sync_copy(iid_hbm.at[pl.ds(wid * NCHUNK, NCHUNK)], iidx)
    pltpu.sync_copy(w_hbm, wvec)
    pltpu.sync_copy(b_hbm, bvec)

    copies = []
    for j in range(NCHUNK):
        copies.append(pltpu.async_copy(
            utab_hbm.at[uidx.at[j]], urows.at[pl.ds(j * CHUNK, CHUNK)], sem))
        copies.append(pltpu.async_copy(
            itab_hbm.at[iidx.at[j]], vrows.at[pl.ds(j * CHUNK, CHUNK)], sem))
    for c in copies:
        c.wait()

    iota = lax.broadcasted_iota(jnp.int32, (16,), 0)
    bias = bvec[...]
    wvals = [wvec[pl.ds(j * 16, 16)] for j in range(D // 16)]

    def group_body(g, carry):
        acc = bias
        for r in range(16):
            row = g * 16 + r
            t = (urows[row, pl.ds(0, 16)] * vrows[row, pl.ds(0, 16)]) * wvals[0]
            for j in range(1, D // 16):
                t = t + (urows[row, pl.ds(j * 16, 16)]
                         * vrows[row, pl.ds(j * 16, 16)]) * wvals[j]
            s = jnp.sum(t)
            acc = jnp.where(iota == r, s, acc)
        outv[pl.ds(g * 16, 16)] = acc
        return carry

    lax.fori_loop(0, GROUPS, group_body, 0)

    pltpu.sync_copy(outv, out_hbm.at[pl.ds(wid * BPW, BPW)])


def kernel(user_id, item_id, user_table, item_table, linear_w, linear_b):
    uid2d = user_id.reshape(NW * NCHUNK, CHUNK).astype(jnp.int32)
    iid2d = item_id.reshape(NW * NCHUNK, CHUNK).astype(jnp.int32)
    w = linear_w.reshape(D)
    b16 = jnp.broadcast_to(linear_b.reshape(()), (16,)).astype(jnp.float32)

    run = functools.partial(
        pl.kernel,
        mesh=plsc.VectorSubcoreMesh(core_axis_name="c", subcore_axis_name="s"),
        out_type=jax.ShapeDtypeStruct((B,), jnp.float32),
        compiler_params=pltpu.CompilerParams(
            needs_layout_passes=False, use_tc_tiling_on_sc=False),
        scratch_types=[
            pltpu.VMEM((NCHUNK, CHUNK), jnp.int32),
            pltpu.VMEM((NCHUNK, CHUNK), jnp.int32),
            pltpu.VMEM((BPW, D), jnp.float32),
            pltpu.VMEM((BPW, D), jnp.float32),
            pltpu.VMEM((D,), jnp.float32),
            pltpu.VMEM((16,), jnp.float32),
            pltpu.VMEM((BPW,), jnp.float32),
            pltpu.SemaphoreType.DMA,
        ],
    )(_gmf_kernel)

    return run(user_table, item_table, uid2d, iid2d, w, b16)


# speedup vs baseline: 1.6228x; 1.6228x over previous
import functools

import jax
import jax.numpy as jnp
from jax import lax
from jax.experimental import pallas as pl
from jax.experimental.pallas import tpu as pltpu
from jax.experimental.pallas import tpu_sc as plsc

B = 16384
D = 64
NC = 2
NS = 16
NW = NC * NS
BPW = B // NW
CHUNK = 128
NCHUNK = BPW // CHUNK
CGROUPS = CHUNK // 16


def _gmf_kernel(utab_hbm, itab_hbm, uid_hbm, iid_hbm, w_hbm, b_hbm, out_hbm,
                uids, iids, ubuf, vbuf, wvec, bvec, outv, sem0, sem1):
    wid = lax.axis_index("s") * NC + lax.axis_index("c")
    sems = (sem0, sem1)

    pltpu.sync_copy(uid_hbm.at[pl.ds(wid * BPW, BPW)], uids)
    pltpu.sync_copy(iid_hbm.at[pl.ds(wid * BPW, BPW)], iids)
    pltpu.sync_copy(w_hbm, wvec)
    pltpu.sync_copy(b_hbm, bvec)

    def issue_chunk(k, p):
        def body(g, carry):
            uvec = uids[pl.ds(k * CHUNK + g * 16, 16)]
            ivec = iids[pl.ds(k * CHUNK + g * 16, 16)]
            for l in range(16):
                row = g * 16 + l
                pltpu.async_copy(utab_hbm.at[uvec[l]], ubuf.at[p, row], sems[p])
                pltpu.async_copy(itab_hbm.at[ivec[l]], vbuf.at[p, row], sems[p])
            return carry
        lax.fori_loop(0, CGROUPS, body, 0)

    def drain_chunk(p):
        pltpu.make_async_copy(
            utab_hbm.at[pl.ds(0, CHUNK)], ubuf.at[p], sems[p]).wait()
        pltpu.make_async_copy(
            itab_hbm.at[pl.ds(0, CHUNK)], vbuf.at[p], sems[p]).wait()

    iota = lax.broadcasted_iota(jnp.int32, (16,), 0)
    bias = bvec[...]
    bscal = bias[0]
    wvals = [wvec[pl.ds(j * 16, 16)] for j in range(D // 16)]

    def compute_chunk(k, p):
        def group_body(g, carry):
            acc = bias
            for r in range(16):
                row = g * 16 + r
                t = (ubuf[p, row, pl.ds(0, 16)]
                     * vbuf[p, row, pl.ds(0, 16)]) * wvals[0]
                for j in range(1, D // 16):
                    t = t + (ubuf[p, row, pl.ds(j * 16, 16)]
                             * vbuf[p, row, pl.ds(j * 16, 16)]) * wvals[j]
                s = jnp.sum(t) + bscal
                acc = jnp.where(iota == r, s, acc)
            outv[pl.ds(k * CHUNK + g * 16, 16)] = acc
            return carry
        lax.fori_loop(0, CGROUPS, group_body, 0)

    issue_chunk(0, 0)
    for k in range(NCHUNK):
        p = k % 2
        if k + 1 < NCHUNK:
            issue_chunk(k + 1, 1 - p)
        drain_chunk(p)
        compute_chunk(k, p)

    pltpu.sync_copy(outv, out_hbm.at[pl.ds(wid * BPW, BPW)])


def kernel(user_id, item_id, user_table, item_table, linear_w, linear_b):
    uid = user_id.astype(jnp.int32)
    iid = item_id.astype(jnp.int32)
    w = linear_w.reshape(D)
    b16 = jnp.broadcast_to(linear_b.reshape(()), (16,)).astype(jnp.float32)

    run = functools.partial(
        pl.kernel,
        mesh=plsc.VectorSubcoreMesh(core_axis_name="c", subcore_axis_name="s"),
        out_type=jax.ShapeDtypeStruct((B,), jnp.float32),
        compiler_params=pltpu.CompilerParams(needs_layout_passes=False),
        scratch_types=[
            pltpu.VMEM((BPW,), jnp.int32),
            pltpu.VMEM((BPW,), jnp.int32),
            pltpu.VMEM((2, CHUNK, D), jnp.float32),
            pltpu.VMEM((2, CHUNK, D), jnp.float32),
            pltpu.VMEM((D,), jnp.float32),
            pltpu.VMEM((16,), jnp.float32),
            pltpu.VMEM((BPW,), jnp.float32),
            pltpu.SemaphoreType.DMA,
            pltpu.SemaphoreType.DMA,
        ],
    )(_gmf_kernel)

    return run(user_table, item_table, uid, iid, w, b16)


# speedup vs baseline: 1.6437x; 1.0128x over previous
import functools

import jax
import jax.numpy as jnp
from jax import lax
from jax.experimental import pallas as pl
from jax.experimental.pallas import tpu as pltpu
from jax.experimental.pallas import tpu_sc as plsc

B = 16384
D = 64
NC = 2
NS = 16
NW = NC * NS
BPW = B // NW
CHUNK = 32
NCHUNK = BPW // CHUNK
NSLOT = 8
CGROUPS = CHUNK // 16


def _gmf_kernel(utab_hbm, itab_hbm, uid_hbm, iid_hbm, w_hbm, b_hbm, out_hbm,
                uids, iids, ubuf, vbuf, wvec, bvec, outv, sems):
    wid = lax.axis_index("s") * NC + lax.axis_index("c")

    pltpu.sync_copy(uid_hbm.at[pl.ds(wid * BPW, BPW)], uids)
    pltpu.sync_copy(iid_hbm.at[pl.ds(wid * BPW, BPW)], iids)
    pltpu.sync_copy(w_hbm, wvec)
    pltpu.sync_copy(b_hbm, bvec)

    def issue_chunk(k, p):
        def body(g, carry):
            uvec = uids[pl.ds(k * CHUNK + g * 16, 16)]
            ivec = iids[pl.ds(k * CHUNK + g * 16, 16)]
            for l in range(16):
                row = g * 16 + l
                pltpu.async_copy(
                    utab_hbm.at[uvec[l]], ubuf.at[p, row], sems.at[p])
                pltpu.async_copy(
                    itab_hbm.at[ivec[l]], vbuf.at[p, row], sems.at[p])
            return carry
        lax.fori_loop(0, CGROUPS, body, 0)

    def drain_chunk(p):
        pltpu.make_async_copy(
            utab_hbm.at[pl.ds(0, CHUNK)], ubuf.at[p], sems.at[p]).wait()
        pltpu.make_async_copy(
            itab_hbm.at[pl.ds(0, CHUNK)], vbuf.at[p], sems.at[p]).wait()

    iota = lax.broadcasted_iota(jnp.int32, (16,), 0)
    bias = bvec[...]
    bscal = bias[0]
    wvals = [wvec[pl.ds(j * 16, 16)] for j in range(D // 16)]

    def compute_chunk(k, p):
        def group_body(g, carry):
            acc = bias
            for r in range(16):
                row = g * 16 + r
                t = (ubuf[p, row, pl.ds(0, 16)]
                     * vbuf[p, row, pl.ds(0, 16)]) * wvals[0]
                for j in range(1, D // 16):
                    t = t + (ubuf[p, row, pl.ds(j * 16, 16)]
                             * vbuf[p, row, pl.ds(j * 16, 16)]) * wvals[j]
                s = jnp.sum(t) + bscal
                acc = jnp.where(iota == r, s, acc)
            outv[pl.ds(k * CHUNK + g * 16, 16)] = acc
            return carry
        lax.fori_loop(0, CGROUPS, group_body, 0)

    def prologue(k, carry):
        issue_chunk(k, k)
        return carry

    lax.fori_loop(0, NSLOT, prologue, 0)

    def steady(k, carry):
        p = k % NSLOT
        drain_chunk(p)
        compute_chunk(k, p)

        @pl.when(k + NSLOT < NCHUNK)
        def _():
            issue_chunk(k + NSLOT, p)
        return carry

    lax.fori_loop(0, NCHUNK, steady, 0)

    pltpu.sync_copy(outv, out_hbm.at[pl.ds(wid * BPW, BPW)])


def kernel(user_id, item_id, user_table, item_table, linear_w, linear_b):
    uid = user_id.astype(jnp.int32)
    iid = item_id.astype(jnp.int32)
    w = linear_w.reshape(D)
    b16 = jnp.broadcast_to(linear_b.reshape(()), (16,)).astype(jnp.float32)

    run = functools.partial(
        pl.kernel,
        mesh=plsc.VectorSubcoreMesh(core_axis_name="c", subcore_axis_name="s"),
        out_type=jax.ShapeDtypeStruct((B,), jnp.float32),
        compiler_params=pltpu.CompilerParams(needs_layout_passes=False),
        scratch_types=[
            pltpu.VMEM((BPW,), jnp.int32),
            pltpu.VMEM((BPW,), jnp.int32),
            pltpu.VMEM((NSLOT, CHUNK, D), jnp.float32),
            pltpu.VMEM((NSLOT, CHUNK, D), jnp.float32),
            pltpu.VMEM((D,), jnp.float32),
            pltpu.VMEM((16,), jnp.float32),
            pltpu.VMEM((BPW,), jnp.float32),
            pltpu.SemaphoreType.DMA((NSLOT,)),
        ],
    )(_gmf_kernel)

    return run(user_table, item_table, uid, iid, w, b16)
